# baseline (speedup 1.0000x reference)
import functools

import jax
import jax.numpy as jnp
from jax import lax
from jax.experimental import pallas as pl
from jax.experimental.pallas import tpu as pltpu
from jax.experimental.pallas import tpu_sc as plsc

_NUM_CORES = 2
_NUM_SUBCORES = 16
_NUM_WORKERS = _NUM_CORES * _NUM_SUBCORES
_CHUNK = 256
_NBUF = 4


@functools.lru_cache(maxsize=None)
def _make_lookup(n_idx, embed_dim):
    assert n_idx % (_NUM_WORKERS * _CHUNK) == 0
    n_chunks = n_idx // (_NUM_WORKERS * _CHUNK)
    assert n_chunks % _NBUF == 0 and n_chunks // _NBUF >= 2
    n_outer = n_chunks // _NBUF
    per_w = n_chunks * _CHUNK
    mesh = plsc.VectorSubcoreMesh(core_axis_name="c", subcore_axis_name="s")

    def body(table_hbm, idx_hbm, out_hbm, idx_v, rows_v, sem_g, sem_w):
        wid = lax.axis_index("s") * _NUM_CORES + lax.axis_index("c")
        base = wid * per_w
        pltpu.sync_copy(idx_hbm.at[wid], idx_v)

        def gather_start(g, slot):
            pltpu.async_copy(
                table_hbm.at[idx_v.at[pl.ds(g * _CHUNK, _CHUNK)]],
                rows_v.at[slot],
                sem_g.at[slot],
            )

        def gather_wait(slot):
            pltpu.make_async_copy(
                table_hbm.at[idx_v.at[pl.ds(0, _CHUNK)]],
                rows_v.at[slot],
                sem_g.at[slot],
            ).wait()

        def write_start(g, slot):
            pltpu.async_copy(
                rows_v.at[slot],
                out_hbm.at[pl.ds(base + g * _CHUNK, _CHUNK)],
                sem_w.at[slot],
            )

        def write_wait(slot):
            pltpu.make_async_copy(
                rows_v.at[slot],
                out_hbm.at[pl.ds(base, _CHUNK)],
                sem_w.at[slot],
            ).wait()

        for b in range(_NBUF - 1):
            gather_start(b, b)

        def block(g0, first, last):
            for b in range(_NBUF):
                g = g0 + b
                gather_wait(b)
                write_start(g, b)
                if last and b > 0:
                    continue
                b2 = (b - 1) % _NBUF
                if not (first and b == 0):
                    write_wait(b2)
                gather_start(g + _NBUF - 1, b2)

        block(0, first=True, last=False)

        def outer(i, carry):
            block(i * _NBUF, first=False, last=False)
            return carry

        lax.fori_loop(1, n_outer - 1, outer, 0)
        block((n_outer - 1) * _NBUF, first=False, last=True)

        for b in range(_NBUF):
            write_wait(b)

    return pl.kernel(
        body,
        out_type=jax.ShapeDtypeStruct((n_idx, embed_dim), jnp.float32),
        mesh=mesh,
        scratch_types=[
            pltpu.VMEM((n_chunks * _CHUNK,), jnp.int32),
            pltpu.VMEM((_NBUF, _CHUNK, embed_dim), jnp.float32),
            pltpu.SemaphoreType.DMA((_NBUF,)),
            pltpu.SemaphoreType.DMA((_NBUF,)),
        ],
        compiler_params=pltpu.CompilerParams(use_tc_tiling_on_sc=False),
    )


def kernel(token_ids, embedding):
    b, l = token_ids.shape
    n_idx = b * l
    embed_dim = embedding.shape[1]
    idx = token_ids.reshape(_NUM_WORKERS, -1).astype(jnp.int32)
    out = _make_lookup(n_idx, embed_dim)(embedding, idx)
    return out.reshape(b, l, embed_dim)


# speedup vs baseline: 1.2245x; 1.2245x over previous
import functools

import jax
import jax.numpy as jnp
from jax import lax
from jax.experimental import pallas as pl
from jax.experimental.pallas import tpu as pltpu
from jax.experimental.pallas import tpu_sc as plsc

_NUM_CORES = 2
_NUM_SUBCORES = 16
_NUM_WORKERS = _NUM_CORES * _NUM_SUBCORES
_CHUNK = 128
_NBUF = 4
_LANES = 128


@functools.lru_cache(maxsize=None)
def _make_lookup(n_idx, embed_dim):
    assert n_idx % (_NUM_WORKERS * _CHUNK) == 0
    n_chunks = n_idx // (_NUM_WORKERS * _CHUNK)
    assert n_chunks % _NBUF == 0 and n_chunks // _NBUF >= 2
    n_outer = n_chunks // _NBUF
    per_w = n_chunks * _CHUNK
    mesh = plsc.VectorSubcoreMesh(core_axis_name="c", subcore_axis_name="s")

    def body(table_hbm, idx_hbm, out_hbm, idx_v, rows_v, sem_g, sem_w):
        wid = lax.axis_index("s") * _NUM_CORES + lax.axis_index("c")
        base = wid * per_w
        pltpu.sync_copy(idx_hbm.at[wid], idx_v)

        def gather_start(g, slot):
            pltpu.async_copy(
                table_hbm.at[idx_v.at[g]], rows_v.at[slot], sem_g.at[slot]
            )

        def gather_wait(slot):
            pltpu.make_async_copy(
                table_hbm.at[idx_v.at[0]], rows_v.at[slot], sem_g.at[slot]
            ).wait()

        def write_start(g, slot):
            pltpu.async_copy(
                rows_v.at[slot],
                out_hbm.at[pl.ds(base + g * _CHUNK, _CHUNK)],
                sem_w.at[slot],
            )

        def write_wait(slot):
            pltpu.make_async_copy(
                rows_v.at[slot],
                out_hbm.at[pl.ds(base, _CHUNK)],
                sem_w.at[slot],
            ).wait()

        for b in range(_NBUF - 1):
            gather_start(b, b)

        def block(g0, first, last):
            for b in range(_NBUF):
                g = g0 + b
                gather_wait(b)
                write_start(g, b)
                if last and b > 0:
                    continue
                b2 = (b - 1) % _NBUF
                if not (first and b == 0):
                    write_wait(b2)
                gather_start(g + _NBUF - 1, b2)

        block(0, first=True, last=False)

        def outer(i, carry):
            block(i * _NBUF, first=False, last=False)
            return carry

        lax.fori_loop(1, n_outer - 1, outer, 0)
        block((n_outer - 1) * _NBUF, first=False, last=True)

        for b in range(_NBUF):
            write_wait(b)

    return pl.kernel(
        body,
        out_type=jax.ShapeDtypeStruct((n_idx, _LANES), jnp.float32),
        mesh=mesh,
        scratch_types=[
            pltpu.VMEM((n_chunks, _CHUNK), jnp.int32),
            pltpu.VMEM((_NBUF, _CHUNK, _LANES), jnp.float32),
            pltpu.SemaphoreType.DMA((_NBUF,)),
            pltpu.SemaphoreType.DMA((_NBUF,)),
        ],
        compiler_params=pltpu.CompilerParams(use_tc_tiling_on_sc=True),
    )


def kernel(token_ids, embedding):
    b, l = token_ids.shape
    n_idx = b * l
    embed_dim = embedding.shape[1]
    table_p = jnp.pad(embedding, ((0, 0), (0, _LANES - embed_dim)))
    idx = token_ids.reshape(_NUM_WORKERS, -1, _CHUNK).astype(jnp.int32)
    out = _make_lookup(n_idx, embed_dim)(table_p, idx)
    out = lax.slice(out, (0, 0), (n_idx, embed_dim))
    return out.reshape(b, l, embed_dim)
